# Initial kernel scaffold; baseline (speedup 1.0000x reference)
import jax
import jax.numpy as jnp
from jax.experimental import pallas as pl


def kernel(event_ids, word_ids, event_table, word_table):
    raise NotImplementedError("write your pallas kernel here")


# speedup vs baseline: 2.2509x; 2.2509x over previous
import functools

import jax
import jax.numpy as jnp
from jax import lax
from jax.experimental import pallas as pl
from jax.experimental.pallas import tpu as pltpu
from jax.experimental.pallas import tpu_sc as plsc

NC = 2
NS = 16
NW = NC * NS
D = 32
CH = 256


def _make_sc_lookup(B, L, EV, WV):
    BL = B * L
    b_per_w = B // NW
    n_half = b_per_w // CH
    mesh = plsc.VectorSubcoreMesh(core_axis_name="c", subcore_axis_name="s")

    @functools.partial(
        pl.kernel,
        mesh=mesh,
        out_type=jax.ShapeDtypeStruct((L, 2 * D, B), jnp.float32),
        compiler_params=pltpu.CompilerParams(needs_layout_passes=False),
        scratch_types=[
            pltpu.VMEM((CH,), jnp.int32),
            pltpu.VMEM((CH,), jnp.int32),
            pltpu.VMEM((CH,), jnp.int32),
            pltpu.VMEM((CH,), jnp.int32),
            pltpu.VMEM((CH, 128), jnp.float32),
            pltpu.VMEM((CH, 128), jnp.float32),
            pltpu.VMEM((D, CH), jnp.float32),
            pltpu.VMEM((D, CH), jnp.float32),
            pltpu.SemaphoreType.DMA,
            pltpu.SemaphoreType.DMA,
        ],
    )
    def lookup(ev_idx, wo_idx, ev_tab, wo_tab, out,
               ie_v, iw_v, re_v, rw_v, ge_v, gw_v, ve_v, vw_v, sem_e, sem_w):
        wid = lax.axis_index("s") * NC + lax.axis_index("c")
        b0 = wid * b_per_w

        def chunk(l, h):
            off = l * B + b0 + h * CH
            pltpu.sync_copy(ev_idx.at[pl.ds(off, CH)], ie_v)
            pltpu.sync_copy(wo_idx.at[pl.ds(off, CH)], iw_v)

            def blkids(j, carry):
                ie = ie_v[pl.ds(j * 16, 16)]
                iw = iw_v[pl.ds(j * 16, 16)]
                re_v[pl.ds(j * 16, 16)] = lax.shift_right_logical(ie, 2)
                rw_v[pl.ds(j * 16, 16)] = lax.shift_right_logical(iw, 2)
                return carry

            lax.fori_loop(0, CH // 16, blkids, 0)
            ce = pltpu.async_copy(ev_tab.at[re_v], ge_v, sem_e)
            cw = pltpu.async_copy(wo_tab.at[rw_v], gw_v, sem_w)
            ce.wait()
            cw.wait()

            def select(j, carry):
                rows = lax.iota(jnp.int32, 16) + j * 16
                ce16 = (ie_v[pl.ds(j * 16, 16)] & 3) * D
                cw16 = (iw_v[pl.ds(j * 16, 16)] & 3) * D
                for d in range(D):
                    ve_v[d, pl.ds(j * 16, 16)] = plsc.load_gather(
                        ge_v, [rows, ce16 + d])
                    vw_v[d, pl.ds(j * 16, 16)] = plsc.load_gather(
                        gw_v, [rows, cw16 + d])
                return carry

            lax.fori_loop(0, CH // 16, select, 0)
            pltpu.sync_copy(ve_v, out.at[l, pl.ds(0, D), pl.ds(b0 + h * CH, CH)])
            pltpu.sync_copy(vw_v, out.at[l, pl.ds(D, D), pl.ds(b0 + h * CH, CH)])

        def l_loop(l, carry):
            def h_loop(h, carry2):
                chunk(l, h)
                return carry2
            lax.fori_loop(0, n_half, h_loop, 0)
            return carry

        lax.fori_loop(0, L, l_loop, 0)

    return lookup


def kernel(event_ids, word_ids, event_table, word_table):
    B, L = event_ids.shape
    EV, _ = event_table.shape
    WV, _ = word_table.shape
    ev_idx = event_ids.T.reshape(B * L).astype(jnp.int32)
    wo_idx = word_ids.T.reshape(B * L).astype(jnp.int32)
    ev4 = event_table.reshape(EV // 4, 4 * D)
    wo4 = word_table.reshape(WV // 4, 4 * D)
    out = _make_sc_lookup(B, L, EV, WV)(ev_idx, wo_idx, ev4, wo4)
    return out.transpose(2, 0, 1)


# speedup vs baseline: 2.9022x; 1.2894x over previous
import functools

import jax
import jax.numpy as jnp
from jax import lax
from jax.experimental import pallas as pl
from jax.experimental.pallas import tpu as pltpu
from jax.experimental.pallas import tpu_sc as plsc

NC = 2
NS = 16
NW = NC * NS
D = 32
CH = 256


def _make_sc_lookup(B, L, EV, WV):
    BL = B * L
    b_per_w = B // NW
    n_half = b_per_w // CH
    mesh = plsc.VectorSubcoreMesh(core_axis_name="c", subcore_axis_name="s")

    @functools.partial(
        pl.kernel,
        mesh=mesh,
        out_type=jax.ShapeDtypeStruct((L, 2 * D, B), jnp.float32),
        compiler_params=pltpu.CompilerParams(needs_layout_passes=False),
        scratch_types=[
            pltpu.VMEM((CH,), jnp.int32),
            pltpu.VMEM((CH,), jnp.int32),
            pltpu.VMEM((CH,), jnp.int32),
            pltpu.VMEM((CH,), jnp.int32),
            pltpu.VMEM((CH, 128), jnp.float32),
            pltpu.VMEM((CH, 128), jnp.float32),
            pltpu.VMEM((D, CH), jnp.float32),
            pltpu.VMEM((D, CH), jnp.float32),
            pltpu.SemaphoreType.DMA,
            pltpu.SemaphoreType.DMA,
        ],
    )
    def lookup(ev_idx, wo_idx, ev_tab, wo_tab, out,
               ie_v, iw_v, re_v, rw_v, ge_v, gw_v, ve_v, vw_v, sem_e, sem_w):
        wid = lax.axis_index("s") * NC + lax.axis_index("c")
        b0 = wid * b_per_w

        def chunk(l, h):
            off = l * B + b0 + h * CH
            pltpu.sync_copy(ev_idx.at[pl.ds(off, CH)], ie_v)
            pltpu.sync_copy(wo_idx.at[pl.ds(off, CH)], iw_v)

            def blkids(j, carry):
                ie = ie_v[pl.ds(j * 16, 16)]
                iw = iw_v[pl.ds(j * 16, 16)]
                re_v[pl.ds(j * 16, 16)] = lax.shift_right_logical(ie, 2)
                rw_v[pl.ds(j * 16, 16)] = lax.shift_right_logical(iw, 2)
                return carry

            lax.fori_loop(0, CH // 16, blkids, 0)
            ce = pltpu.async_copy(ev_tab.at[re_v], ge_v, sem_e)
            cw = pltpu.async_copy(wo_tab.at[rw_v], gw_v, sem_w)
            ce.wait()
            cw.wait()

            def select(j, carry):
                rows = lax.iota(jnp.int32, 16) + j * 16
                ce16 = (ie_v[pl.ds(j * 16, 16)] & 3) * D
                cw16 = (iw_v[pl.ds(j * 16, 16)] & 3) * D
                ev_g = [plsc.load_gather(ge_v, [rows, ce16 + d])
                        for d in range(D)]
                wo_g = [plsc.load_gather(gw_v, [rows, cw16 + d])
                        for d in range(D)]
                for d in range(D):
                    ve_v[d, pl.ds(j * 16, 16)] = ev_g[d]
                    vw_v[d, pl.ds(j * 16, 16)] = wo_g[d]
                return carry

            lax.fori_loop(0, CH // 16, select, 0)
            pltpu.sync_copy(ve_v, out.at[l, pl.ds(0, D), pl.ds(b0 + h * CH, CH)])
            pltpu.sync_copy(vw_v, out.at[l, pl.ds(D, D), pl.ds(b0 + h * CH, CH)])

        def l_loop(l, carry):
            def h_loop(h, carry2):
                chunk(l, h)
                return carry2
            lax.fori_loop(0, n_half, h_loop, 0)
            return carry

        lax.fori_loop(0, L, l_loop, 0)

    return lookup


def kernel(event_ids, word_ids, event_table, word_table):
    B, L = event_ids.shape
    EV, _ = event_table.shape
    WV, _ = word_table.shape
    ev_idx = event_ids.T.reshape(B * L).astype(jnp.int32)
    wo_idx = word_ids.T.reshape(B * L).astype(jnp.int32)
    ev4 = event_table.reshape(EV // 4, 4 * D)
    wo4 = word_table.reshape(WV // 4, 4 * D)
    out = _make_sc_lookup(B, L, EV, WV)(ev_idx, wo_idx, ev4, wo4)
    return out.transpose(2, 0, 1)


# speedup vs baseline: 2.9141x; 1.0041x over previous
import functools

import jax
import jax.numpy as jnp
from jax import lax
from jax.experimental import pallas as pl
from jax.experimental.pallas import tpu as pltpu
from jax.experimental.pallas import tpu_sc as plsc

NC = 2
NS = 16
NW = NC * NS
D = 32
CH = 256


def _make_sc_lookup(B, L, EV, WV):
    BL = B * L
    b_per_w = B // NW
    n_half = b_per_w // CH
    mesh = plsc.VectorSubcoreMesh(core_axis_name="c", subcore_axis_name="s")

    @functools.partial(
        pl.kernel,
        mesh=mesh,
        out_type=jax.ShapeDtypeStruct((L, 2 * D, B), jnp.float32),
        compiler_params=pltpu.CompilerParams(needs_layout_passes=False),
        scratch_types=[
            pltpu.VMEM((CH,), jnp.int32),
            pltpu.VMEM((CH,), jnp.int32),
            pltpu.VMEM((CH,), jnp.int32),
            pltpu.VMEM((CH,), jnp.int32),
            pltpu.VMEM((CH, 128), jnp.float32),
            pltpu.VMEM((CH, 128), jnp.float32),
            pltpu.VMEM((D, CH), jnp.float32),
            pltpu.VMEM((D, CH), jnp.float32),
            pltpu.SemaphoreType.DMA,
            pltpu.SemaphoreType.DMA,
        ],
    )
    def lookup(ev_idx, wo_idx, ev_tab, wo_tab, out,
               ie_v, iw_v, re_v, rw_v, ge_v, gw_v, ve_v, vw_v, sem_e, sem_w):
        wid = lax.axis_index("s") * NC + lax.axis_index("c")
        b0 = wid * b_per_w

        def chunk(l, h):
            off = l * B + b0 + h * CH
            pltpu.sync_copy(ev_idx.at[pl.ds(off, CH)], ie_v)
            pltpu.sync_copy(wo_idx.at[pl.ds(off, CH)], iw_v)

            def blkids(j, carry):
                ie = ie_v[pl.ds(j * 16, 16)]
                iw = iw_v[pl.ds(j * 16, 16)]
                re_v[pl.ds(j * 16, 16)] = lax.shift_right_logical(ie, 2)
                rw_v[pl.ds(j * 16, 16)] = lax.shift_right_logical(iw, 2)
                return carry

            lax.fori_loop(0, CH // 16, blkids, 0)
            ce = pltpu.async_copy(ev_tab.at[re_v], ge_v, sem_e)
            cw = pltpu.async_copy(wo_tab.at[rw_v], gw_v, sem_w)
            ce.wait()
            cw.wait()

            def select(j, carry):
                rows = lax.iota(jnp.int32, 16) + j * 16
                ce16 = (ie_v[pl.ds(j * 16, 16)] & 3) * D
                cw16 = (iw_v[pl.ds(j * 16, 16)] & 3) * D
                for half in range(2):
                    ev_g = [plsc.load_gather(ge_v, [rows, ce16 + (16 * half + k)])
                            for k in range(16)]
                    for k in range(16):
                        ve_v[16 * half + k, pl.ds(j * 16, 16)] = ev_g[k]
                    wo_g = [plsc.load_gather(gw_v, [rows, cw16 + (16 * half + k)])
                            for k in range(16)]
                    for k in range(16):
                        vw_v[16 * half + k, pl.ds(j * 16, 16)] = wo_g[k]
                return carry

            lax.fori_loop(0, CH // 16, select, 0)
            pltpu.sync_copy(ve_v, out.at[l, pl.ds(0, D), pl.ds(b0 + h * CH, CH)])
            pltpu.sync_copy(vw_v, out.at[l, pl.ds(D, D), pl.ds(b0 + h * CH, CH)])

        def l_loop(l, carry):
            def h_loop(h, carry2):
                chunk(l, h)
                return carry2
            lax.fori_loop(0, n_half, h_loop, 0)
            return carry

        lax.fori_loop(0, L, l_loop, 0)

    return lookup


def kernel(event_ids, word_ids, event_table, word_table):
    B, L = event_ids.shape
    EV, _ = event_table.shape
    WV, _ = word_table.shape
    ev_idx = event_ids.T.reshape(B * L).astype(jnp.int32)
    wo_idx = word_ids.T.reshape(B * L).astype(jnp.int32)
    ev4 = event_table.reshape(EV // 4, 4 * D)
    wo4 = word_table.reshape(WV // 4, 4 * D)
    out = _make_sc_lookup(B, L, EV, WV)(ev_idx, wo_idx, ev4, wo4)
    return out.transpose(2, 0, 1)
